# baseline (speedup 1.0000x reference)
import jax
import jax.numpy as jnp
from jax import lax
from jax.experimental import pallas as pl
from jax.experimental.pallas import tpu as pltpu
from jax.experimental.pallas import tpu_sc as plsc

LN2 = 0.6931471805599453

NUM_FRAME = 1000
NUM_PARAM = 3
B, L = 16384, 200
N_IDS = B * L
N_OUT = N_IDS * NUM_PARAM

NC, NS = 2, 16
NW = NC * NS

LB = L // 8
CB = 1024
NCHUNK_B = B // CB
NCHUNKS = LB * NCHUNK_B
CHUNK_IDS = 8 * CB
PER_W = NCHUNKS // NW
NVEC = CHUNK_IDS // 16

TBL_RAW = NUM_FRAME * NUM_PARAM
TBL_RAW_PAD = 3072
TBL_PAD = 1024


def _sc_body(ids_hbm, vars_hbm, out_hbm,
             vars_buf, tbl0, tbl1, tbl2,
             ids_buf0, ids_buf1, ob00, ob01, ob02, ob10, ob11, ob12,
             ob20, ob21, ob22,
             in_sem0, in_sem1, out_sem0, out_sem1, out_sem2, tail_sem):
    ids_bufs = (ids_buf0, ids_buf1)
    out_bufs = ((ob00, ob01, ob02), (ob10, ob11, ob12), (ob20, ob21, ob22))
    in_sems = (in_sem0, in_sem1)
    out_sems = (out_sem0, out_sem1, out_sem2)
    wid = lax.axis_index("s") * NC + lax.axis_index("c")

    def _early_in(i, buf, sem):
        c = wid + NW * i
        l_hi = c >> 4
        cb = c & 15
        ibase = l_hi * (8 * B) + cb * CB
        descs = [
            pltpu.make_async_copy(
                ids_hbm.at[pl.ds(ibase + l_lo * B, CB)],
                buf.at[pl.ds(l_lo * CB, CB)],
                sem)
            for l_lo in range(8)
        ]
        for d in descs:
            d.start()
        return descs

    early_in0 = _early_in(0, ids_buf0, in_sem0)
    early_in1 = _early_in(1, ids_buf1, in_sem1)

    pltpu.sync_copy(vars_hbm, vars_buf.at[pl.ds(0, TBL_RAW)])
    iota = lax.iota(jnp.int32, 16)
    iota3 = iota * 3

    @plsc.parallel_loop(0, 63, unroll=4)
    def tbl_body(i):
        idx = i * 48 + iota3
        x0 = plsc.load_gather(vars_buf, [idx])
        x1 = plsc.load_gather(vars_buf, [idx + 1])
        x2 = plsc.load_gather(vars_buf, [idx + 2])
        tbl0[pl.ds(i * 16, 16)] = jnp.exp(LN2 * x0)
        tbl1[pl.ds(i * 16, 16)] = jnp.exp(LN2 * x1)
        tbl2[pl.ds(i * 16, 16)] = jnp.exp(LN2 * x2)

    def in_descs(c, buf, sem):
        l_hi = c >> 4
        cb = c & 15
        ibase = l_hi * (8 * B) + cb * CB
        return [
            pltpu.make_async_copy(
                ids_hbm.at[pl.ds(ibase + l_lo * B, CB)],
                buf.at[pl.ds(l_lo * CB, CB)],
                sem)
            for l_lo in range(8)
        ]

    def out_descs(c, bufs, sem):
        l_hi = c >> 4
        cb = c & 15
        obase = l_hi * (8 * B) + cb * (8 * CB)
        return [
            pltpu.make_async_copy(
                bufs[k],
                out_hbm.at[pl.ds(k * N_IDS + obase, CHUNK_IDS)],
                sem)
            for k in range(NUM_PARAM)
        ]

    def compute(ids_buf, obufs, cb_words, nvec):
        ob0, ob1, ob2 = obufs

        @plsc.parallel_loop(0, nvec, unroll=8)
        def vec_body(w):
            q = ((w >> 3) & 7) * cb_words + (w >> 6) * 128 + (w & 7) * 16
            p = w * 16
            v = ids_buf[pl.ds(q, 16)]
            ob0[pl.ds(p, 16)] = plsc.load_gather(tbl0, [v])
            ob1[pl.ds(p, 16)] = plsc.load_gather(tbl1, [v])
            ob2[pl.ds(p, 16)] = plsc.load_gather(tbl2, [v])

    tail_c = NW * PER_W + (wid >> 1)
    tail_h = wid & 1
    HCB = CB // 2
    t_l_hi = tail_c >> 4
    t_cb = tail_c & 15
    t_ibase = t_l_hi * (8 * B) + t_cb * CB + tail_h * HCB
    tail_in = [
        pltpu.make_async_copy(
            ids_hbm.at[pl.ds(t_ibase + l_lo * B, HCB)],
            ids_bufs[PER_W % 2].at[pl.ds(l_lo * HCB, HCB)],
            tail_sem)
        for l_lo in range(8)
    ]
    tail_set = PER_W % 3
    t_obase = t_l_hi * (8 * B) + t_cb * (8 * CB) + tail_h * (8 * HCB)
    tail_out = [
        pltpu.make_async_copy(
            out_bufs[tail_set][k].at[pl.ds(0, 8 * HCB)],
            out_hbm.at[pl.ds(k * N_IDS + t_obase, 8 * HCB)],
            tail_sem)
        for k in range(NUM_PARAM)
    ]

    def chunk_c(i):
        return wid + NW * i

    pending_in = {0: early_in0, 1: early_in1}
    pending_out = {}
    for i in range(PER_W):
        for d in pending_in.pop(i):
            d.wait()
        if i - 3 in pending_out:
            for d in pending_out.pop(i - 3):
                d.wait()
        compute(ids_bufs[i % 2], out_bufs[i % 3], CB, NVEC)
        if i + 2 < PER_W:
            pending_in[i + 2] = in_descs(
                chunk_c(i + 2), ids_bufs[(i + 2) % 2], in_sems[(i + 2) % 2])
            for d in pending_in[i + 2]:
                d.start()
        elif i + 2 == PER_W:
            for d in tail_in:
                d.start()
        pending_out[i] = out_descs(chunk_c(i), out_bufs[i % 3], out_sems[i % 3])
        for d in pending_out[i]:
            d.start()

    for d in tail_in:
        d.wait()
    if PER_W - 3 in pending_out:
        for d in pending_out.pop(PER_W - 3):
            d.wait()
    compute(ids_bufs[PER_W % 2], out_bufs[tail_set], HCB, NVEC // 2)
    for d in tail_out:
        d.start()

    for i, descs in sorted(pending_out.items()):
        for d in descs:
            d.wait()
    for d in tail_out:
        d.wait()


@jax.jit
def kernel(ids, vars_):
    ids_flat = jnp.transpose(ids, (1, 2, 0)).reshape(N_IDS)
    vars_flat = vars_.reshape(TBL_RAW)
    mesh = plsc.VectorSubcoreMesh(
        core_axis_name="c", subcore_axis_name="s", num_cores=NC, num_subcores=NS
    )
    out_flat = pl.kernel(
        _sc_body,
        out_type=jax.ShapeDtypeStruct((N_OUT,), jnp.float32),
        mesh=mesh,
        scratch_types=[
            pltpu.VMEM((TBL_RAW_PAD,), jnp.float32),
            pltpu.VMEM((TBL_PAD,), jnp.float32),
            pltpu.VMEM((TBL_PAD,), jnp.float32),
            pltpu.VMEM((TBL_PAD,), jnp.float32),
            pltpu.VMEM((CHUNK_IDS,), jnp.int32),
            pltpu.VMEM((CHUNK_IDS,), jnp.int32),
            pltpu.VMEM((CHUNK_IDS,), jnp.float32),
            pltpu.VMEM((CHUNK_IDS,), jnp.float32),
            pltpu.VMEM((CHUNK_IDS,), jnp.float32),
            pltpu.VMEM((CHUNK_IDS,), jnp.float32),
            pltpu.VMEM((CHUNK_IDS,), jnp.float32),
            pltpu.VMEM((CHUNK_IDS,), jnp.float32),
            pltpu.VMEM((CHUNK_IDS,), jnp.float32),
            pltpu.VMEM((CHUNK_IDS,), jnp.float32),
            pltpu.VMEM((CHUNK_IDS,), jnp.float32),
            pltpu.SemaphoreType.DMA,
            pltpu.SemaphoreType.DMA,
            pltpu.SemaphoreType.DMA,
            pltpu.SemaphoreType.DMA,
            pltpu.SemaphoreType.DMA,
            pltpu.SemaphoreType.DMA,
        ],
        compiler_params=pltpu.CompilerParams(needs_layout_passes=False),
    )(ids_flat, vars_flat)
    x = out_flat.reshape(NUM_PARAM, LB, B // 128, 8, 128)
    return x.transpose(2, 4, 1, 3, 0).reshape(B, L, NUM_PARAM)


# speedup vs baseline: 1.0030x; 1.0030x over previous
import jax
import jax.numpy as jnp
from jax import lax
from jax.experimental import pallas as pl
from jax.experimental.pallas import tpu as pltpu
from jax.experimental.pallas import tpu_sc as plsc

LN2 = 0.6931471805599453

NUM_FRAME = 1000
NUM_PARAM = 3
B, L = 16384, 200
N_IDS = B * L
N_OUT = N_IDS * NUM_PARAM

NC, NS = 2, 16
NW = NC * NS

LB = L // 8
CB = 1024
NCHUNK_B = B // CB
NCHUNKS = LB * NCHUNK_B
CHUNK_IDS = 8 * CB
PER_W = NCHUNKS // NW
NVEC = CHUNK_IDS // 16

TBL_RAW = NUM_FRAME * NUM_PARAM
TBL_RAW_PAD = 3072
TBL_PAD = 1024


def _sc_body(ids_hbm, vars_hbm, out_hbm,
             vars_buf, tbl0, tbl1, tbl2,
             ids_buf0, ids_buf1, ob00, ob01, ob02, ob10, ob11, ob12,
             ob20, ob21, ob22,
             in_sem0, in_sem1, out_sem0, out_sem1, out_sem2, tail_sem):
    ids_bufs = (ids_buf0, ids_buf1)
    out_bufs = ((ob00, ob01, ob02), (ob10, ob11, ob12), (ob20, ob21, ob22))
    in_sems = (in_sem0, in_sem1)
    out_sems = (out_sem0, out_sem1, out_sem2)
    wid = lax.axis_index("s") * NC + lax.axis_index("c")

    def _early_in(i, buf, sem):
        c = wid + NW * i
        l_hi = c >> 4
        cb = c & 15
        ibase = l_hi * (8 * B) + cb * CB
        descs = [
            pltpu.make_async_copy(
                ids_hbm.at[pl.ds(ibase + l_lo * B, CB)],
                buf.at[pl.ds(l_lo * CB, CB)],
                sem)
            for l_lo in range(8)
        ]
        for d in descs:
            d.start()
        return descs

    early_in0 = _early_in(0, ids_buf0, in_sem0)
    early_in1 = _early_in(1, ids_buf1, in_sem1)

    pltpu.sync_copy(vars_hbm, vars_buf.at[pl.ds(0, TBL_RAW)])
    iota = lax.iota(jnp.int32, 16)
    iota3 = iota * 3

    @plsc.parallel_loop(0, 63, unroll=4)
    def tbl_body(i):
        idx = i * 48 + iota3
        x0 = plsc.load_gather(vars_buf, [idx])
        x1 = plsc.load_gather(vars_buf, [idx + 1])
        x2 = plsc.load_gather(vars_buf, [idx + 2])
        tbl0[pl.ds(i * 16, 16)] = jnp.exp(LN2 * x0)
        tbl1[pl.ds(i * 16, 16)] = jnp.exp(LN2 * x1)
        tbl2[pl.ds(i * 16, 16)] = jnp.exp(LN2 * x2)

    def in_descs(c, buf, sem):
        l_hi = c >> 4
        cb = c & 15
        ibase = l_hi * (8 * B) + cb * CB
        return [
            pltpu.make_async_copy(
                ids_hbm.at[pl.ds(ibase + l_lo * B, CB)],
                buf.at[pl.ds(l_lo * CB, CB)],
                sem)
            for l_lo in range(8)
        ]

    def out_descs(c, bufs, sem):
        l_hi = c >> 4
        cb = c & 15
        obase = l_hi * (8 * B) + cb * (8 * CB)
        return [
            pltpu.make_async_copy(
                bufs[k],
                out_hbm.at[pl.ds(k * N_IDS + obase, CHUNK_IDS)],
                sem)
            for k in range(NUM_PARAM)
        ]

    def compute(ids_buf, obufs, cb_words, nvec):
        ob0, ob1, ob2 = obufs

        @plsc.parallel_loop(0, nvec, unroll=8)
        def vec_body(w):
            q = ((w >> 3) & 7) * cb_words + (w >> 6) * 128 + (w & 7) * 16
            p = w * 16
            v = ids_buf[pl.ds(q, 16)]
            ob0[pl.ds(p, 16)] = plsc.load_gather(tbl0, [v])
            ob1[pl.ds(p, 16)] = plsc.load_gather(tbl1, [v])
            ob2[pl.ds(p, 16)] = plsc.load_gather(tbl2, [v])

    tail_c = NW * PER_W + (wid >> 1)
    tail_h = wid & 1
    HCB = CB // 2
    t_l_hi = tail_c >> 4
    t_cb = tail_c & 15
    t_ibase = t_l_hi * (8 * B) + t_cb * CB + tail_h * HCB
    tail_in = [
        pltpu.make_async_copy(
            ids_hbm.at[pl.ds(t_ibase + l_lo * B, HCB)],
            ids_bufs[PER_W % 2].at[pl.ds(l_lo * HCB, HCB)],
            tail_sem)
        for l_lo in range(8)
    ]
    tail_set = PER_W % 3
    t_obase = t_l_hi * (8 * B) + t_cb * (8 * CB) + tail_h * (8 * HCB)
    tail_out = [
        pltpu.make_async_copy(
            out_bufs[tail_set][k].at[pl.ds(0, 8 * HCB)],
            out_hbm.at[pl.ds(k * N_IDS + t_obase, 8 * HCB)],
            tail_sem)
        for k in range(NUM_PARAM)
    ]

    def chunk_c(i):
        return wid + NW * i

    pending_in = {0: early_in0, 1: early_in1}
    pending_out = {}
    for i in range(PER_W):
        for d in pending_in.pop(i):
            d.wait()
        if i - 3 in pending_out:
            for d in pending_out.pop(i - 3):
                d.wait()
        compute(ids_bufs[i % 2], out_bufs[i % 3], CB, NVEC)
        if i + 2 < PER_W:
            pending_in[i + 2] = in_descs(
                chunk_c(i + 2), ids_bufs[(i + 2) % 2], in_sems[(i + 2) % 2])
            for d in pending_in[i + 2]:
                d.start()
        elif i + 2 == PER_W:
            for d in tail_in:
                d.start()
        pending_out[i] = out_descs(chunk_c(i), out_bufs[i % 3], out_sems[i % 3])
        for d in pending_out[i]:
            d.start()

    for d in tail_in:
        d.wait()
    if PER_W - 3 in pending_out:
        for d in pending_out.pop(PER_W - 3):
            d.wait()
    compute(ids_bufs[PER_W % 2], out_bufs[tail_set], HCB, NVEC // 2)
    for d in tail_out:
        d.start()

    for i, descs in sorted(pending_out.items()):
        for d in descs:
            d.wait()
    for d in tail_out:
        d.wait()


@jax.jit
def kernel(ids, vars_):
    ids_flat = jnp.transpose(ids, (1, 2, 0)).reshape(N_IDS)
    vars_flat = vars_.reshape(TBL_RAW)
    mesh = plsc.VectorSubcoreMesh(
        core_axis_name="c", subcore_axis_name="s", num_cores=NC, num_subcores=NS
    )
    out_flat = pl.kernel(
        _sc_body,
        out_type=jax.ShapeDtypeStruct((N_OUT,), jnp.float32),
        mesh=mesh,
        scratch_types=[
            pltpu.VMEM((TBL_RAW_PAD,), jnp.float32),
            pltpu.VMEM((TBL_PAD,), jnp.float32),
            pltpu.VMEM((TBL_PAD,), jnp.float32),
            pltpu.VMEM((TBL_PAD,), jnp.float32),
            pltpu.VMEM((CHUNK_IDS,), jnp.int32),
            pltpu.VMEM((CHUNK_IDS,), jnp.int32),
            pltpu.VMEM((CHUNK_IDS,), jnp.float32),
            pltpu.VMEM((CHUNK_IDS,), jnp.float32),
            pltpu.VMEM((CHUNK_IDS,), jnp.float32),
            pltpu.VMEM((CHUNK_IDS,), jnp.float32),
            pltpu.VMEM((CHUNK_IDS,), jnp.float32),
            pltpu.VMEM((CHUNK_IDS,), jnp.float32),
            pltpu.VMEM((CHUNK_IDS,), jnp.float32),
            pltpu.VMEM((CHUNK_IDS,), jnp.float32),
            pltpu.VMEM((CHUNK_IDS,), jnp.float32),
            pltpu.SemaphoreType.DMA,
            pltpu.SemaphoreType.DMA,
            pltpu.SemaphoreType.DMA,
            pltpu.SemaphoreType.DMA,
            pltpu.SemaphoreType.DMA,
            pltpu.SemaphoreType.DMA,
        ],
        compiler_params=pltpu.CompilerParams(
            needs_layout_passes=False,
            disable_bounds_checks=True,
            disable_semaphore_checks=True,
        ),
    )(ids_flat, vars_flat)
    x = out_flat.reshape(NUM_PARAM, LB, B // 128, 8, 128)
    return x.transpose(2, 4, 1, 3, 0).reshape(B, L, NUM_PARAM)
